# Initial kernel scaffold; baseline (speedup 1.0000x reference)
import jax
import jax.numpy as jnp
from jax.experimental import pallas as pl


def kernel(x, edge_index, edge_type, Wconv, bconv, bn_gamma, bn_beta, W_e1, b_e1, W_e2, b_e2, Wp1a, bp1a, Wp1b, bp1b, Wp2a, bp2a, Wp2b, bp2b):
    raise NotImplementedError("write your pallas kernel here")


# speedup vs baseline: 12.7337x; 12.7337x over previous
import functools

import jax
import jax.numpy as jnp
from jax import lax
from jax.experimental import pallas as pl
from jax.experimental.pallas import tpu as pltpu
from jax.experimental.pallas import tpu_sc as plsc

_N = 10000
_E = 320000
_D = 128
_NC = 2
_NS = 16
_NW = _NC * _NS
_EPT = _E // _NW
_CH = 128
_NFULL = _EPT // _CH
_REM = _EPT - _NFULL * _CH
_N2 = 2 * _N
_N2P = 20480
_WPS = _N2P // _NS
_RPS = 624
_RTAIL = _N - _RPS * _NS


def _f32(shape):
    return jax.ShapeDtypeStruct(shape, jnp.float32)


def _i32(shape):
    return jax.ShapeDtypeStruct(shape, jnp.int32)


def _make_sc_prep(interpret=False):
    mesh = plsc.VectorSubcoreMesh(core_axis_name="c", subcore_axis_name="s")

    @functools.partial(
        pl.kernel,
        out_type=(_i32((_E,)), _i32((_E,)), _f32((_NC * _N2P,))),
        mesh=mesh,
        scratch_types=(
            pltpu.VMEM((_CH,), jnp.int32),
            pltpu.VMEM((_CH,), jnp.int32),
            pltpu.VMEM((_CH,), jnp.int32),
            pltpu.VMEM((_CH,), jnp.int32),
            pltpu.VMEM((_CH,), jnp.int32),
            pltpu.VMEM((_CH,), jnp.float32),
            pltpu.VMEM((_REM,), jnp.int32),
            pltpu.VMEM((_REM,), jnp.int32),
            pltpu.VMEM((_REM,), jnp.int32),
            pltpu.VMEM((_REM,), jnp.int32),
            pltpu.VMEM((_REM,), jnp.int32),
            pltpu.VMEM((_REM,), jnp.float32),
            pltpu.VMEM_SHARED((_N2P,), jnp.float32),
        ),
        compiler_params=pltpu.CompilerParams(needs_layout_passes=False),
        interpret=interpret,
    )
    def prep(src_hbm, dst_hbm, et_hbm, ones_hbm, zw_hbm,
             esrc2_hbm, edst2_hbm, degcnt_hbm,
             src_v, dst_v, et_v, es_v, ed_v, ones_v,
             src_r, dst_r, et_r, es_r, ed_r, ones_r,
             acc_sh):
        c = lax.axis_index("c")
        s = lax.axis_index("s")
        w = c * _NS + s
        zoff = pl.multiple_of(s * _WPS, 8)
        pltpu.sync_copy(zw_hbm, acc_sh.at[pl.ds(zoff, _WPS)])
        pltpu.sync_copy(ones_hbm, ones_v)
        pltpu.sync_copy(ones_hbm.at[pl.ds(0, _REM)], ones_r)
        plsc.subcore_barrier()

        base0 = w * _EPT

        @pl.loop(0, _NFULL)
        def _chunk(ci):
            off = pl.multiple_of(base0 + ci * _CH, 8)
            pltpu.sync_copy(src_hbm.at[pl.ds(off, _CH)], src_v)
            pltpu.sync_copy(dst_hbm.at[pl.ds(off, _CH)], dst_v)
            pltpu.sync_copy(et_hbm.at[pl.ds(off, _CH)], et_v)
            for j in range(_CH // 16):
                sl = pl.ds(16 * j, 16)
                t16 = et_v[sl]
                es_v[sl] = t16 * _N + src_v[sl]
                ed_v[sl] = t16 * _N + dst_v[sl]
            pltpu.sync_copy(es_v, esrc2_hbm.at[pl.ds(off, _CH)])
            pltpu.sync_copy(ed_v, edst2_hbm.at[pl.ds(off, _CH)])
            pltpu.sync_copy(ones_v, acc_sh.at[ed_v], add=True)

        roff = pl.multiple_of(base0 + _NFULL * _CH, 8)
        pltpu.sync_copy(src_hbm.at[pl.ds(roff, _REM)], src_r)
        pltpu.sync_copy(dst_hbm.at[pl.ds(roff, _REM)], dst_r)
        pltpu.sync_copy(et_hbm.at[pl.ds(roff, _REM)], et_r)
        t16 = et_r[...]
        es_r[...] = t16 * _N + src_r[...]
        ed_r[...] = t16 * _N + dst_r[...]
        pltpu.sync_copy(es_r, esrc2_hbm.at[pl.ds(roff, _REM)])
        pltpu.sync_copy(ed_r, edst2_hbm.at[pl.ds(roff, _REM)])
        pltpu.sync_copy(ones_r, acc_sh.at[ed_r], add=True)

        plsc.subcore_barrier()
        doff = pl.multiple_of(s * _WPS, 8)
        doff2 = pl.multiple_of(c * _N2P + s * _WPS, 8)
        pltpu.sync_copy(acc_sh.at[pl.ds(doff, _WPS)],
                        degcnt_hbm.at[pl.ds(doff2, _WPS)])

    return prep


def _make_sc_agg(interpret=False):
    mesh = plsc.VectorSubcoreMesh(core_axis_name="c", subcore_axis_name="s")

    @functools.partial(
        pl.kernel,
        out_type=_f32((_NC, _N, _D)),
        mesh=mesh,
        scratch_types=(
            pltpu.VMEM((_N2P,), jnp.float32),
            pltpu.VMEM((_CH,), jnp.int32),
            pltpu.VMEM((_CH,), jnp.int32),
            pltpu.VMEM((_CH,), jnp.int32),
            pltpu.VMEM((_CH,), jnp.float32),
            pltpu.VMEM((_CH, _D), jnp.float32),
            pltpu.VMEM((_REM,), jnp.int32),
            pltpu.VMEM((_REM,), jnp.int32),
            pltpu.VMEM((_REM,), jnp.int32),
            pltpu.VMEM((_REM,), jnp.float32),
            pltpu.VMEM((_REM, _D), jnp.float32),
            pltpu.VMEM_SHARED((_N, _D), jnp.float32),
        ),
        compiler_params=pltpu.CompilerParams(needs_layout_passes=False),
        interpret=interpret,
    )
    def agg(g_hbm, esrc2_hbm, edst2_hbm, dst_hbm, rs2_hbm, zr_hbm,
            agg2_hbm,
            rs_v, gi_v, di_v, ei_v, s_v, rows_v,
            gi_r, di_r, ei_r, s_r, rows_r,
            acc_sh):
        c = lax.axis_index("c")
        s = lax.axis_index("s")
        w = c * _NS + s
        zoff = pl.multiple_of(s * _RPS, 8)
        pltpu.sync_copy(zr_hbm, acc_sh.at[pl.ds(zoff, _RPS)])

        @pl.when(s == _NS - 1)
        def _ztail():
            pltpu.sync_copy(zr_hbm.at[pl.ds(0, _RTAIL)],
                            acc_sh.at[pl.ds(_NS * _RPS, _RTAIL)])

        pltpu.sync_copy(rs2_hbm, rs_v)
        plsc.subcore_barrier()

        base0 = w * _EPT

        def do_chunk(off, k, gi, di, ei, sv, rows):
            pltpu.sync_copy(esrc2_hbm.at[pl.ds(off, k)], gi)
            pltpu.sync_copy(edst2_hbm.at[pl.ds(off, k)], ei)
            pltpu.sync_copy(dst_hbm.at[pl.ds(off, k)], di)
            pltpu.sync_copy(g_hbm.at[gi], rows)
            for jg in range(k // 16):
                svals = plsc.load_gather(rs_v, [ei[pl.ds(16 * jg, 16)]])
                for l in range(16):
                    e = 16 * jg + l
                    splat = svals.at[jnp.full((16,), l, jnp.int32)].get(
                        mode='promise_in_bounds')
                    for j in range(_D // 16):
                        sl = pl.ds(16 * j, 16)
                        rows[e, sl] = rows[e, sl] * splat
            pltpu.sync_copy(rows, acc_sh.at[di], add=True)

        @pl.loop(0, _NFULL)
        def _chunk(ci):
            off = pl.multiple_of(base0 + ci * _CH, 8)
            do_chunk(off, _CH, gi_v, di_v, ei_v, s_v, rows_v)

        roff = pl.multiple_of(base0 + _NFULL * _CH, 8)
        do_chunk(roff, _REM, gi_r, di_r, ei_r, s_r, rows_r)

        plsc.subcore_barrier()
        pltpu.sync_copy(acc_sh.at[pl.ds(zoff, _RPS)],
                        agg2_hbm.at[c, pl.ds(zoff, _RPS)])

        @pl.when(s == _NS - 1)
        def _dtail():
            pltpu.sync_copy(acc_sh.at[pl.ds(_NS * _RPS, _RTAIL)],
                            agg2_hbm.at[c, pl.ds(_NS * _RPS, _RTAIL)])

    return agg


def _tc_pre_body(x_ref, w0_ref, w1_ref, degcnt_ref, g_ref, rs2_ref):
    deg = degcnt_ref[:_N2P] + degcnt_ref[_N2P:] + 1.0
    rs2 = lax.rsqrt(deg)
    rs2_ref[...] = rs2
    x = x_ref[...]
    h0 = jnp.dot(x, w0_ref[...], preferred_element_type=jnp.float32)
    h1 = jnp.dot(x, w1_ref[...], preferred_element_type=jnp.float32)
    g_ref[:_N] = h0 * rs2[:_N, None]
    g_ref[_N:] = h1 * rs2[_N:_N2, None]


def _bn_from(agg_ref, g_ref, rs2_ref, bsum_ref, gamma_ref, beta_ref):
    rs2 = rs2_ref[...]
    g = g_ref[...]
    pre = (agg_ref[0] + agg_ref[1]
           + g[:_N] * rs2[:_N, None]
           + g[_N:] * rs2[_N:_N2, None]
           + bsum_ref[...])
    mu = jnp.mean(pre, axis=0)
    var = jnp.mean((pre - mu) ** 2, axis=0)
    return gamma_ref[...] * (pre - mu) / jnp.sqrt(var + 1e-5) + beta_ref[...]


def _tc_post_body(agg_ref, g_ref, rs2_ref, bsum_ref, gamma_ref, beta_ref,
                  wn0_ref, wn1_ref, gout_ref):
    h = jnp.maximum(_bn_from(agg_ref, g_ref, rs2_ref, bsum_ref,
                             gamma_ref, beta_ref), 0.0)
    rs2 = rs2_ref[...]
    gout_ref[:_N] = jnp.dot(h, wn0_ref[...],
                            preferred_element_type=jnp.float32) * rs2[:_N, None]
    gout_ref[_N:] = jnp.dot(h, wn1_ref[...],
                            preferred_element_type=jnp.float32) * rs2[_N:_N2, None]


def _l2n(x):
    n = jnp.sqrt(jnp.sum(x * x, axis=1, keepdims=True))
    return x / jnp.maximum(n, 1e-12)


def _tc_heads_body(agg_ref, g_ref, rs2_ref, bsum_ref, gamma_ref, beta_ref,
                   we1_ref, be1_ref, we2_ref, be2_ref,
                   wp1a_ref, bp1a_ref, wp1b_ref, bp1b_ref,
                   wp2a_ref, bp2a_ref, wp2b_ref, bp2b_ref,
                   e1_ref, e2_ref, p1_ref, p2_ref):
    h = _bn_from(agg_ref, g_ref, rs2_ref, bsum_ref, gamma_ref, beta_ref)
    dot = lambda a, b: jnp.dot(a, b, preferred_element_type=jnp.float32)
    e1 = jnp.tanh(dot(h, we1_ref[...]) + be1_ref[...])
    e2 = _l2n(jnp.tanh(dot(h, we2_ref[...]) + be2_ref[...]))
    p1 = _l2n(dot(jnp.maximum(dot(e1, wp1a_ref[...]) + bp1a_ref[...], 0.0),
                  wp1b_ref[...]) + bp1b_ref[...])
    p2 = _l2n(dot(jnp.maximum(dot(e2, wp2a_ref[...]) + bp2a_ref[...], 0.0),
                  wp2b_ref[...]) + bp2b_ref[...])
    e1_ref[...] = e1
    e2_ref[...] = e2
    p1_ref[...] = p1
    p2_ref[...] = p2


def _tc_pre(x, w0, w1, degcnt, interpret=False):
    return pl.pallas_call(
        _tc_pre_body,
        out_shape=(_f32((_N2, _D)), _f32((_N2P,))),
        interpret=interpret,
    )(x, w0, w1, degcnt)


def _tc_post(agg2, g, rs2, bsum, gamma, beta, wn0, wn1, interpret=False):
    return pl.pallas_call(
        _tc_post_body,
        out_shape=_f32((_N2, _D)),
        interpret=interpret,
    )(agg2, g, rs2, bsum, gamma, beta, wn0, wn1)


def _tc_heads(agg2, g, rs2, bsum, gamma, beta, heads, interpret=False):
    return pl.pallas_call(
        _tc_heads_body,
        out_shape=(_f32((_N, _D)),) * 4,
        interpret=interpret,
    )(agg2, g, rs2, bsum, gamma, beta, *heads)


def kernel(x, edge_index, edge_type, Wconv, bconv, bn_gamma, bn_beta,
           W_e1, b_e1, W_e2, b_e2, Wp1a, bp1a, Wp1b, bp1b,
           Wp2a, bp2a, Wp2b, bp2b):
    src = edge_index[0]
    dst = edge_index[1]
    et = edge_type.astype(jnp.int32)
    ones = jnp.ones((_CH,), jnp.float32)
    zwords = jnp.zeros((_WPS,), jnp.float32)
    zrows = jnp.zeros((_RPS, _D), jnp.float32)

    sc_prep = _make_sc_prep()
    sc_agg = _make_sc_agg()

    esrc2, edst2, degcnt = sc_prep(src, dst, et, ones, zwords)
    g, rs2 = _tc_pre(x, Wconv[0, 0], Wconv[0, 1], degcnt)

    for layer in range(3):
        agg2 = sc_agg(g, esrc2, edst2, dst, rs2, zrows)
        bsum = bconv[layer, 0] + bconv[layer, 1]
        if layer < 2:
            g = _tc_post(agg2, g, rs2, bsum, bn_gamma[layer], bn_beta[layer],
                         Wconv[layer + 1, 0], Wconv[layer + 1, 1])
        else:
            heads = (W_e1, b_e1, W_e2, b_e2, Wp1a, bp1a, Wp1b, bp1b,
                     Wp2a, bp2a, Wp2b, bp2b)
            e1, e2, p1, p2 = _tc_heads(agg2, g, rs2, bsum,
                                       bn_gamma[layer], bn_beta[layer], heads)
    return (e1, e2, p1, p2)


# speedup vs baseline: 18.4545x; 1.4493x over previous
import functools

import jax
import jax.numpy as jnp
from jax import lax
from jax.experimental import pallas as pl
from jax.experimental.pallas import tpu as pltpu
from jax.experimental.pallas import tpu_sc as plsc

_N = 10000
_E = 320000
_D = 128
_NC = 2
_NS = 16
_NW = _NC * _NS
_EPT = _E // _NW
_CH = 128
_NFULL = _EPT // _CH
_REM = _EPT - _NFULL * _CH
_N2 = 2 * _N
_N2P = 20480
_WPS = _N2P // _NS
_RPS = 624
_RTAIL = _N - _RPS * _NS


def _f32(shape):
    return jax.ShapeDtypeStruct(shape, jnp.float32)


def _i32(shape):
    return jax.ShapeDtypeStruct(shape, jnp.int32)


def _make_sc_prep(interpret=False):
    mesh = plsc.VectorSubcoreMesh(core_axis_name="c", subcore_axis_name="s")

    @functools.partial(
        pl.kernel,
        out_type=(_i32((_E,)), _i32((_E,)), _f32((_NC * _N2P,))),
        mesh=mesh,
        scratch_types=(
            pltpu.VMEM((_CH,), jnp.int32),
            pltpu.VMEM((_CH,), jnp.int32),
            pltpu.VMEM((_CH,), jnp.int32),
            pltpu.VMEM((_CH,), jnp.int32),
            pltpu.VMEM((_CH,), jnp.int32),
            pltpu.VMEM((_CH,), jnp.float32),
            pltpu.VMEM((_REM,), jnp.int32),
            pltpu.VMEM((_REM,), jnp.int32),
            pltpu.VMEM((_REM,), jnp.int32),
            pltpu.VMEM((_REM,), jnp.int32),
            pltpu.VMEM((_REM,), jnp.int32),
            pltpu.VMEM((_REM,), jnp.float32),
            pltpu.VMEM_SHARED((_N2P,), jnp.float32),
        ),
        compiler_params=pltpu.CompilerParams(needs_layout_passes=False),
        interpret=interpret,
    )
    def prep(src_hbm, dst_hbm, et_hbm, ones_hbm, zw_hbm,
             esrc2_hbm, edst2_hbm, degcnt_hbm,
             src_v, dst_v, et_v, es_v, ed_v, ones_v,
             src_r, dst_r, et_r, es_r, ed_r, ones_r,
             acc_sh):
        c = lax.axis_index("c")
        s = lax.axis_index("s")
        w = c * _NS + s
        zoff = pl.multiple_of(s * _WPS, 8)
        pltpu.sync_copy(zw_hbm, acc_sh.at[pl.ds(zoff, _WPS)])
        pltpu.sync_copy(ones_hbm, ones_v)
        pltpu.sync_copy(ones_hbm.at[pl.ds(0, _REM)], ones_r)
        plsc.subcore_barrier()

        base0 = w * _EPT

        @pl.loop(0, _NFULL)
        def _chunk(ci):
            off = pl.multiple_of(base0 + ci * _CH, 8)
            pltpu.sync_copy(src_hbm.at[pl.ds(off, _CH)], src_v)
            pltpu.sync_copy(dst_hbm.at[pl.ds(off, _CH)], dst_v)
            pltpu.sync_copy(et_hbm.at[pl.ds(off, _CH)], et_v)
            for j in range(_CH // 16):
                sl = pl.ds(16 * j, 16)
                t16 = et_v[sl]
                es_v[sl] = t16 * _N + src_v[sl]
                ed_v[sl] = t16 * _N + dst_v[sl]
            pltpu.sync_copy(es_v, esrc2_hbm.at[pl.ds(off, _CH)])
            pltpu.sync_copy(ed_v, edst2_hbm.at[pl.ds(off, _CH)])
            pltpu.sync_copy(ones_v, acc_sh.at[ed_v], add=True)

        roff = pl.multiple_of(base0 + _NFULL * _CH, 8)
        pltpu.sync_copy(src_hbm.at[pl.ds(roff, _REM)], src_r)
        pltpu.sync_copy(dst_hbm.at[pl.ds(roff, _REM)], dst_r)
        pltpu.sync_copy(et_hbm.at[pl.ds(roff, _REM)], et_r)
        t16 = et_r[...]
        es_r[...] = t16 * _N + src_r[...]
        ed_r[...] = t16 * _N + dst_r[...]
        pltpu.sync_copy(es_r, esrc2_hbm.at[pl.ds(roff, _REM)])
        pltpu.sync_copy(ed_r, edst2_hbm.at[pl.ds(roff, _REM)])
        pltpu.sync_copy(ones_r, acc_sh.at[ed_r], add=True)

        plsc.subcore_barrier()
        doff = pl.multiple_of(s * _WPS, 8)
        doff2 = pl.multiple_of(c * _N2P + s * _WPS, 8)
        pltpu.sync_copy(acc_sh.at[pl.ds(doff, _WPS)],
                        degcnt_hbm.at[pl.ds(doff2, _WPS)])

    return prep


def _make_sc_sgather(interpret=False):
    mesh = plsc.VectorSubcoreMesh(core_axis_name="c", subcore_axis_name="s")

    @functools.partial(
        pl.kernel,
        out_type=_f32((_E,)),
        mesh=mesh,
        scratch_types=(
            pltpu.VMEM((_N2P,), jnp.float32),
            pltpu.VMEM((_CH,), jnp.int32),
            pltpu.VMEM((_CH,), jnp.float32),
        ),
        compiler_params=pltpu.CompilerParams(needs_layout_passes=False),
        interpret=interpret,
    )
    def sgather(edst2_hbm, rs2_hbm, s_hbm, rs_v, ei_v, sv_v):
        c = lax.axis_index("c")
        s = lax.axis_index("s")
        w = c * _NS + s
        pltpu.sync_copy(rs2_hbm, rs_v)
        base0 = w * _EPT

        def do_chunk(off):
            pltpu.sync_copy(edst2_hbm.at[pl.ds(off, _CH)], ei_v)
            for jg in range(_CH // 16):
                sl = pl.ds(16 * jg, 16)
                sv_v[sl] = plsc.load_gather(rs_v, [ei_v[sl]])
            pltpu.sync_copy(sv_v, s_hbm.at[pl.ds(off, _CH)])

        @pl.loop(0, _NFULL)
        def _chunk(ci):
            do_chunk(pl.multiple_of(base0 + ci * _CH, 8))

        do_chunk(pl.multiple_of(base0 + _EPT - _CH, 8))

    return sgather


def _make_sc_agg(interpret=False):
    mesh = plsc.VectorSubcoreMesh(core_axis_name="c", subcore_axis_name="s")

    @functools.partial(
        pl.kernel,
        out_type=_f32((_NC, _N, _D)),
        mesh=mesh,
        scratch_types=(
            pltpu.VMEM((_CH,), jnp.int32),
            pltpu.VMEM((_CH,), jnp.int32),
            pltpu.VMEM((_CH,), jnp.float32),
            pltpu.VMEM((_CH, _D), jnp.float32),
            pltpu.VMEM((_CH,), jnp.int32),
            pltpu.VMEM((_CH,), jnp.int32),
            pltpu.VMEM((_CH,), jnp.float32),
            pltpu.VMEM((_CH, _D), jnp.float32),
            pltpu.VMEM((_CH,), jnp.int32),
            pltpu.VMEM((_CH,), jnp.int32),
            pltpu.VMEM((_CH,), jnp.float32),
            pltpu.VMEM((_CH, _D), jnp.float32),
            pltpu.SemaphoreType.DMA,
            pltpu.SemaphoreType.DMA,
            pltpu.SemaphoreType.DMA,
            pltpu.SemaphoreType.DMA,
            pltpu.SemaphoreType.DMA,
            pltpu.SemaphoreType.DMA,
            pltpu.VMEM_SHARED((_N, _D), jnp.float32),
        ),
        compiler_params=pltpu.CompilerParams(needs_layout_passes=False),
        interpret=interpret,
    )
    def agg(g_hbm, esrc2_hbm, dst_hbm, s_hbm, zr_hbm,
            agg2_hbm,
            gi0, di0, sv0, rows0,
            gi1, di1, sv1, rows1,
            gi2, di2, sv2, rows2,
            gsem0, gsem1, gsem2, ssem0, ssem1, ssem2,
            acc_sh):
        c = lax.axis_index("c")
        s = lax.axis_index("s")
        w = c * _NS + s
        zoff = pl.multiple_of(s * _RPS, 8)
        pltpu.sync_copy(zr_hbm, acc_sh.at[pl.ds(zoff, _RPS)])

        @pl.when(s == _NS - 1)
        def _ztail():
            pltpu.sync_copy(zr_hbm.at[pl.ds(0, _RTAIL)],
                            acc_sh.at[pl.ds(_NS * _RPS, _RTAIL)])

        plsc.subcore_barrier()

        base0 = w * _EPT
        bufs = ((gi0, di0, sv0, rows0, gsem0, ssem0),
                (gi1, di1, sv1, rows1, gsem1, ssem1),
                (gi2, di2, sv2, rows2, gsem2, ssem2))

        def issue(off, b):
            gi, di, sv, rows, gsem, _ = bufs[b]
            pltpu.sync_copy(esrc2_hbm.at[pl.ds(off, _CH)], gi)
            pltpu.sync_copy(dst_hbm.at[pl.ds(off, _CH)], di)
            pltpu.sync_copy(s_hbm.at[pl.ds(off, _CH)], sv)
            pltpu.async_copy(g_hbm.at[gi], rows, gsem)

        def wait_gather(b):
            gi, _, _, rows, gsem, _ = bufs[b]
            pltpu.make_async_copy(g_hbm.at[gi], rows, gsem).wait()

        def wait_scatter(b):
            _, di, _, rows, _, ssem = bufs[b]
            pltpu.make_async_copy(rows, acc_sh.at[di], ssem).wait()

        def scale(b):
            _, _, sv, rows, _, _ = bufs[b]

            @pl.loop(0, _CH // 16)
            def _grp(jg):
                off16 = pl.multiple_of(jg * 16, 16)
                svals = sv[pl.ds(off16, 16)]
                for l in range(16):
                    splat = svals.at[jnp.full((16,), l, jnp.int32)].get(
                        mode='promise_in_bounds')
                    for j in range(_D // 16):
                        sl = pl.ds(16 * j, 16)
                        rows[off16 + l, sl] = rows[off16 + l, sl] * splat

        def start_scatter(b):
            _, di, _, rows, _, ssem = bufs[b]
            pltpu.async_copy(rows, acc_sh.at[di], ssem, add=True)

        n3 = _NFULL // 3
        issue(pl.multiple_of(base0, 8), 0)

        @pl.loop(0, n3)
        def _outer(g3):
            for b in range(3):
                ci = g3 * 3 + b
                bn = (b + 1) % 3
                noff = pl.multiple_of(base0 + (ci + 1) * _CH, 8)
                if b < 2:
                    @pl.when(g3 >= 1)
                    def _w():
                        wait_scatter(bn)
                    issue(noff, bn)
                else:
                    wait_scatter(bn)

                    @pl.when(g3 < n3 - 1)
                    def _i():
                        issue(noff, bn)
                wait_gather(b)
                scale(b)
                start_scatter(b)

        wait_scatter(1)
        wait_scatter(2)

        roff = pl.multiple_of(base0 + _EPT - _CH, 8)
        pltpu.sync_copy(esrc2_hbm.at[pl.ds(roff, _CH)], gi0)
        pltpu.sync_copy(dst_hbm.at[pl.ds(roff, _CH)], di0)
        pltpu.sync_copy(s_hbm.at[pl.ds(roff, _CH)], sv0)
        for jg in range((_CH - _REM) // 16):
            sv0[pl.ds(16 * jg, 16)] = jnp.zeros((16,), jnp.float32)
        pltpu.sync_copy(g_hbm.at[gi0], rows0)
        scale(0)
        pltpu.sync_copy(rows0, acc_sh.at[di0], add=True)

        plsc.subcore_barrier()
        pltpu.sync_copy(acc_sh.at[pl.ds(zoff, _RPS)],
                        agg2_hbm.at[c, pl.ds(zoff, _RPS)])

        @pl.when(s == _NS - 1)
        def _dtail():
            pltpu.sync_copy(acc_sh.at[pl.ds(_NS * _RPS, _RTAIL)],
                            agg2_hbm.at[c, pl.ds(_NS * _RPS, _RTAIL)])

    return agg


def _tc_pre_body(x_ref, w0_ref, w1_ref, degcnt_ref, g_ref, rs2_ref):
    deg = degcnt_ref[:_N2P] + degcnt_ref[_N2P:] + 1.0
    rs2 = lax.rsqrt(deg)
    rs2_ref[...] = rs2
    x = x_ref[...]
    h0 = jnp.dot(x, w0_ref[...], preferred_element_type=jnp.float32)
    h1 = jnp.dot(x, w1_ref[...], preferred_element_type=jnp.float32)
    g_ref[:_N] = h0 * rs2[:_N, None]
    g_ref[_N:] = h1 * rs2[_N:_N2, None]


def _bn_from(agg_ref, g_ref, rs2_ref, bsum_ref, gamma_ref, beta_ref):
    rs2 = rs2_ref[...]
    g = g_ref[...]
    pre = (agg_ref[0] + agg_ref[1]
           + g[:_N] * rs2[:_N, None]
           + g[_N:] * rs2[_N:_N2, None]
           + bsum_ref[...])
    mu = jnp.mean(pre, axis=0)
    var = jnp.mean((pre - mu) ** 2, axis=0)
    return gamma_ref[...] * (pre - mu) / jnp.sqrt(var + 1e-5) + beta_ref[...]


def _tc_post_body(agg_ref, g_ref, rs2_ref, bsum_ref, gamma_ref, beta_ref,
                  wn0_ref, wn1_ref, gout_ref):
    h = jnp.maximum(_bn_from(agg_ref, g_ref, rs2_ref, bsum_ref,
                             gamma_ref, beta_ref), 0.0)
    rs2 = rs2_ref[...]
    gout_ref[:_N] = jnp.dot(h, wn0_ref[...],
                            preferred_element_type=jnp.float32) * rs2[:_N, None]
    gout_ref[_N:] = jnp.dot(h, wn1_ref[...],
                            preferred_element_type=jnp.float32) * rs2[_N:_N2, None]


def _l2n(x):
    n = jnp.sqrt(jnp.sum(x * x, axis=1, keepdims=True))
    return x / jnp.maximum(n, 1e-12)


def _tc_heads_body(agg_ref, g_ref, rs2_ref, bsum_ref, gamma_ref, beta_ref,
                   we1_ref, be1_ref, we2_ref, be2_ref,
                   wp1a_ref, bp1a_ref, wp1b_ref, bp1b_ref,
                   wp2a_ref, bp2a_ref, wp2b_ref, bp2b_ref,
                   e1_ref, e2_ref, p1_ref, p2_ref):
    h = _bn_from(agg_ref, g_ref, rs2_ref, bsum_ref, gamma_ref, beta_ref)
    dot = lambda a, b: jnp.dot(a, b, preferred_element_type=jnp.float32)
    e1 = jnp.tanh(dot(h, we1_ref[...]) + be1_ref[...])
    e2 = _l2n(jnp.tanh(dot(h, we2_ref[...]) + be2_ref[...]))
    p1 = _l2n(dot(jnp.maximum(dot(e1, wp1a_ref[...]) + bp1a_ref[...], 0.0),
                  wp1b_ref[...]) + bp1b_ref[...])
    p2 = _l2n(dot(jnp.maximum(dot(e2, wp2a_ref[...]) + bp2a_ref[...], 0.0),
                  wp2b_ref[...]) + bp2b_ref[...])
    e1_ref[...] = e1
    e2_ref[...] = e2
    p1_ref[...] = p1
    p2_ref[...] = p2


def _tc_pre(x, w0, w1, degcnt, interpret=False):
    return pl.pallas_call(
        _tc_pre_body,
        out_shape=(_f32((_N2, _D)), _f32((_N2P,))),
        interpret=interpret,
    )(x, w0, w1, degcnt)


def _tc_post(agg2, g, rs2, bsum, gamma, beta, wn0, wn1, interpret=False):
    return pl.pallas_call(
        _tc_post_body,
        out_shape=_f32((_N2, _D)),
        interpret=interpret,
    )(agg2, g, rs2, bsum, gamma, beta, wn0, wn1)


def _tc_heads(agg2, g, rs2, bsum, gamma, beta, heads, interpret=False):
    return pl.pallas_call(
        _tc_heads_body,
        out_shape=(_f32((_N, _D)),) * 4,
        interpret=interpret,
    )(agg2, g, rs2, bsum, gamma, beta, *heads)


def kernel(x, edge_index, edge_type, Wconv, bconv, bn_gamma, bn_beta,
           W_e1, b_e1, W_e2, b_e2, Wp1a, bp1a, Wp1b, bp1b,
           Wp2a, bp2a, Wp2b, bp2b):
    src = edge_index[0]
    dst = edge_index[1]
    et = edge_type.astype(jnp.int32)
    ones = jnp.ones((_CH,), jnp.float32)
    zwords = jnp.zeros((_WPS,), jnp.float32)
    zrows = jnp.zeros((_RPS, _D), jnp.float32)

    sc_prep = _make_sc_prep()
    sc_sgather = _make_sc_sgather()
    sc_agg = _make_sc_agg()

    esrc2, edst2, degcnt = sc_prep(src, dst, et, ones, zwords)
    g, rs2 = _tc_pre(x, Wconv[0, 0], Wconv[0, 1], degcnt)
    sedge = sc_sgather(edst2, rs2)

    for layer in range(3):
        agg2 = sc_agg(g, esrc2, dst, sedge, zrows)
        bsum = bconv[layer, 0] + bconv[layer, 1]
        if layer < 2:
            g = _tc_post(agg2, g, rs2, bsum, bn_gamma[layer], bn_beta[layer],
                         Wconv[layer + 1, 0], Wconv[layer + 1, 1])
        else:
            heads = (W_e1, b_e1, W_e2, b_e2, Wp1a, bp1a, Wp1b, bp1b,
                     Wp2a, bp2a, Wp2b, bp2b)
            e1, e2, p1, p2 = _tc_heads(agg2, g, rs2, bsum,
                                       bn_gamma[layer], bn_beta[layer], heads)
    return (e1, e2, p1, p2)


# speedup vs baseline: 22.7369x; 1.2321x over previous
import functools

import jax
import jax.numpy as jnp
from jax import lax
from jax.experimental import pallas as pl
from jax.experimental.pallas import tpu as pltpu
from jax.experimental.pallas import tpu_sc as plsc

_N = 10000
_E = 320000
_D = 128
_NC = 2
_NS = 16
_NW = _NC * _NS
_EPT = _E // _NW
_CH = 128
_NFULL = _EPT // _CH
_REM = _EPT - _NFULL * _CH
_N2 = 2 * _N
_N2P = 20480
_WPS = _N2P // _NS
_RPS = 624
_RTAIL = _N - _RPS * _NS


def _f32(shape):
    return jax.ShapeDtypeStruct(shape, jnp.float32)


def _i32(shape):
    return jax.ShapeDtypeStruct(shape, jnp.int32)


def _make_sc_prep(interpret=False):
    mesh = plsc.VectorSubcoreMesh(core_axis_name="c", subcore_axis_name="s")

    @functools.partial(
        pl.kernel,
        out_type=(_i32((_E,)), _i32((_E,)), _f32((_NC * _N2P,))),
        mesh=mesh,
        scratch_types=(
            pltpu.VMEM((_CH,), jnp.int32),
            pltpu.VMEM((_CH,), jnp.int32),
            pltpu.VMEM((_CH,), jnp.int32),
            pltpu.VMEM((_CH,), jnp.int32),
            pltpu.VMEM((_CH,), jnp.int32),
            pltpu.VMEM((_CH,), jnp.float32),
            pltpu.VMEM((_REM,), jnp.int32),
            pltpu.VMEM((_REM,), jnp.int32),
            pltpu.VMEM((_REM,), jnp.int32),
            pltpu.VMEM((_REM,), jnp.int32),
            pltpu.VMEM((_REM,), jnp.int32),
            pltpu.VMEM((_REM,), jnp.float32),
            pltpu.VMEM_SHARED((_N2P,), jnp.float32),
        ),
        compiler_params=pltpu.CompilerParams(needs_layout_passes=False),
        interpret=interpret,
    )
    def prep(src_hbm, dst_hbm, et_hbm, ones_hbm, zw_hbm,
             esrc2_hbm, edst2_hbm, degcnt_hbm,
             src_v, dst_v, et_v, es_v, ed_v, ones_v,
             src_r, dst_r, et_r, es_r, ed_r, ones_r,
             acc_sh):
        c = lax.axis_index("c")
        s = lax.axis_index("s")
        w = c * _NS + s
        zoff = pl.multiple_of(s * _WPS, 8)
        pltpu.sync_copy(zw_hbm, acc_sh.at[pl.ds(zoff, _WPS)])
        pltpu.sync_copy(ones_hbm, ones_v)
        pltpu.sync_copy(ones_hbm.at[pl.ds(0, _REM)], ones_r)
        plsc.subcore_barrier()

        base0 = w * _EPT

        @pl.loop(0, _NFULL)
        def _chunk(ci):
            off = pl.multiple_of(base0 + ci * _CH, 8)
            pltpu.sync_copy(src_hbm.at[pl.ds(off, _CH)], src_v)
            pltpu.sync_copy(dst_hbm.at[pl.ds(off, _CH)], dst_v)
            pltpu.sync_copy(et_hbm.at[pl.ds(off, _CH)], et_v)
            for j in range(_CH // 16):
                sl = pl.ds(16 * j, 16)
                t16 = et_v[sl]
                es_v[sl] = t16 * _N + src_v[sl]
                ed_v[sl] = t16 * _N + dst_v[sl]
            pltpu.sync_copy(es_v, esrc2_hbm.at[pl.ds(off, _CH)])
            pltpu.sync_copy(ed_v, edst2_hbm.at[pl.ds(off, _CH)])
            pltpu.sync_copy(ones_v, acc_sh.at[ed_v], add=True)

        roff = pl.multiple_of(base0 + _NFULL * _CH, 8)
        pltpu.sync_copy(src_hbm.at[pl.ds(roff, _REM)], src_r)
        pltpu.sync_copy(dst_hbm.at[pl.ds(roff, _REM)], dst_r)
        pltpu.sync_copy(et_hbm.at[pl.ds(roff, _REM)], et_r)
        t16 = et_r[...]
        es_r[...] = t16 * _N + src_r[...]
        ed_r[...] = t16 * _N + dst_r[...]
        pltpu.sync_copy(es_r, esrc2_hbm.at[pl.ds(roff, _REM)])
        pltpu.sync_copy(ed_r, edst2_hbm.at[pl.ds(roff, _REM)])
        pltpu.sync_copy(ones_r, acc_sh.at[ed_r], add=True)

        plsc.subcore_barrier()
        doff = pl.multiple_of(s * _WPS, 8)
        doff2 = pl.multiple_of(c * _N2P + s * _WPS, 8)
        pltpu.sync_copy(acc_sh.at[pl.ds(doff, _WPS)],
                        degcnt_hbm.at[pl.ds(doff2, _WPS)])

    return prep


def _make_sc_sgather(interpret=False):
    mesh = plsc.VectorSubcoreMesh(core_axis_name="c", subcore_axis_name="s")

    @functools.partial(
        pl.kernel,
        out_type=_f32((_E,)),
        mesh=mesh,
        scratch_types=(
            pltpu.VMEM((_N2P,), jnp.float32),
            pltpu.VMEM((_CH,), jnp.int32),
            pltpu.VMEM((_CH,), jnp.float32),
        ),
        compiler_params=pltpu.CompilerParams(needs_layout_passes=False),
        interpret=interpret,
    )
    def sgather(edst2_hbm, rs2_hbm, s_hbm, rs_v, ei_v, sv_v):
        c = lax.axis_index("c")
        s = lax.axis_index("s")
        w = c * _NS + s
        pltpu.sync_copy(rs2_hbm, rs_v)
        base0 = w * _EPT

        def do_chunk(off):
            pltpu.sync_copy(edst2_hbm.at[pl.ds(off, _CH)], ei_v)
            for jg in range(_CH // 16):
                sl = pl.ds(16 * jg, 16)
                sv_v[sl] = plsc.load_gather(rs_v, [ei_v[sl]])
            pltpu.sync_copy(sv_v, s_hbm.at[pl.ds(off, _CH)])

        @pl.loop(0, _NFULL)
        def _chunk(ci):
            do_chunk(pl.multiple_of(base0 + ci * _CH, 8))

        do_chunk(pl.multiple_of(base0 + _EPT - _CH, 8))

    return sgather


def _make_sc_agg(interpret=False):
    mesh = plsc.VectorSubcoreMesh(core_axis_name="c", subcore_axis_name="s")

    @functools.partial(
        pl.kernel,
        out_type=_f32((_NC, _N, _D)),
        mesh=mesh,
        scratch_types=(
            pltpu.VMEM((_CH,), jnp.int32),
            pltpu.VMEM((_CH,), jnp.int32),
            pltpu.VMEM((_CH,), jnp.float32),
            pltpu.VMEM((_CH, _D), jnp.float32),
            pltpu.VMEM((_CH,), jnp.int32),
            pltpu.VMEM((_CH,), jnp.int32),
            pltpu.VMEM((_CH,), jnp.float32),
            pltpu.VMEM((_CH, _D), jnp.float32),
            pltpu.VMEM((_CH,), jnp.int32),
            pltpu.VMEM((_CH,), jnp.int32),
            pltpu.VMEM((_CH,), jnp.float32),
            pltpu.VMEM((_CH, _D), jnp.float32),
            pltpu.SemaphoreType.DMA,
            pltpu.SemaphoreType.DMA,
            pltpu.SemaphoreType.DMA,
            pltpu.SemaphoreType.DMA,
            pltpu.SemaphoreType.DMA,
            pltpu.SemaphoreType.DMA,
            pltpu.SemaphoreType.DMA,
            pltpu.SemaphoreType.DMA,
            pltpu.SemaphoreType.DMA,
            pltpu.VMEM_SHARED((_N, _D), jnp.float32),
        ),
        compiler_params=pltpu.CompilerParams(needs_layout_passes=False),
        interpret=interpret,
    )
    def agg(g_hbm, esrc2_hbm, dst_hbm, s_hbm, zr_hbm,
            agg2_hbm,
            gi0, di0, sv0, rows0,
            gi1, di1, sv1, rows1,
            gi2, di2, sv2, rows2,
            gsem0, gsem1, gsem2, ssem0, ssem1, ssem2,
            isem0, isem1, isem2,
            acc_sh):
        c = lax.axis_index("c")
        s = lax.axis_index("s")
        w = c * _NS + s
        zoff = pl.multiple_of(s * _RPS, 8)
        pltpu.sync_copy(zr_hbm, acc_sh.at[pl.ds(zoff, _RPS)])

        @pl.when(s == _NS - 1)
        def _ztail():
            pltpu.sync_copy(zr_hbm.at[pl.ds(0, _RTAIL)],
                            acc_sh.at[pl.ds(_NS * _RPS, _RTAIL)])

        plsc.subcore_barrier()

        base0 = w * _EPT
        bufs = ((gi0, di0, sv0, rows0, gsem0, ssem0, isem0),
                (gi1, di1, sv1, rows1, gsem1, ssem1, isem1),
                (gi2, di2, sv2, rows2, gsem2, ssem2, isem2))

        def issue_idx(off, b):
            gi, di, sv, _, _, _, isem = bufs[b]
            pltpu.async_copy(esrc2_hbm.at[pl.ds(off, _CH)], gi, isem)
            pltpu.async_copy(dst_hbm.at[pl.ds(off, _CH)], di, isem)
            pltpu.async_copy(s_hbm.at[pl.ds(off, _CH)], sv, isem)

        def wait_idx(b):
            gi, di, sv, _, _, _, isem = bufs[b]
            z = pl.ds(0, _CH)
            pltpu.make_async_copy(esrc2_hbm.at[z], gi, isem).wait()
            pltpu.make_async_copy(dst_hbm.at[z], di, isem).wait()
            pltpu.make_async_copy(s_hbm.at[z], sv, isem).wait()

        def start_gather(b):
            gi, _, _, rows, gsem, _, _ = bufs[b]
            pltpu.async_copy(g_hbm.at[gi], rows, gsem)

        def wait_gather(b):
            gi, _, _, rows, gsem, _, _ = bufs[b]
            pltpu.make_async_copy(g_hbm.at[gi], rows, gsem).wait()

        def wait_scatter(b):
            _, di, _, rows, _, ssem, _ = bufs[b]
            pltpu.make_async_copy(rows, acc_sh.at[di], ssem).wait()

        def scale(b):
            _, _, sv, rows, _, _, _ = bufs[b]

            @pl.loop(0, _CH // 16)
            def _grp(jg):
                off16 = pl.multiple_of(jg * 16, 16)
                svals = sv[pl.ds(off16, 16)]
                for l in range(16):
                    splat = svals.at[jnp.full((16,), l, jnp.int32)].get(
                        mode='promise_in_bounds')
                    for j in range(_D // 16):
                        sl = pl.ds(16 * j, 16)
                        rows[off16 + l, sl] = rows[off16 + l, sl] * splat

        def start_scatter(b):
            _, di, _, rows, _, ssem, _ = bufs[b]
            pltpu.async_copy(rows, acc_sh.at[di], ssem, add=True)

        def coff(ci):
            return pl.multiple_of(base0 + ci * _CH, 8)

        n3 = _NFULL // 3
        issue_idx(coff(0), 0)
        issue_idx(coff(1), 1)
        wait_idx(0)
        start_gather(0)

        @pl.loop(0, n3)
        def _outer(g3):
            for b in range(3):
                ci = g3 * 3 + b
                b1 = (b + 1) % 3
                b2 = (b + 2) % 3
                if b == 0:
                    @pl.when(g3 >= 1)
                    def _w0():
                        wait_scatter(b2)
                    issue_idx(coff(ci + 2), b2)
                else:
                    @pl.when(g3 < n3 - 1)
                    def _w1():
                        wait_scatter(b2)
                        issue_idx(coff(ci + 2), b2)
                if b < 2:
                    wait_idx(b1)
                    start_gather(b1)
                else:
                    @pl.when(g3 < n3 - 1)
                    def _g1():
                        wait_idx(b1)
                        start_gather(b1)
                wait_gather(b)
                scale(b)
                start_scatter(b)

        wait_scatter(0)
        wait_scatter(1)
        wait_scatter(2)

        roff = pl.multiple_of(base0 + _EPT - _CH, 8)
        issue_idx(roff, 0)
        wait_idx(0)
        for jg in range((_CH - _REM) // 16):
            sv0[pl.ds(16 * jg, 16)] = jnp.zeros((16,), jnp.float32)
        pltpu.sync_copy(g_hbm.at[gi0], rows0)
        scale(0)
        pltpu.sync_copy(rows0, acc_sh.at[di0], add=True)

        plsc.subcore_barrier()
        pltpu.sync_copy(acc_sh.at[pl.ds(zoff, _RPS)],
                        agg2_hbm.at[c, pl.ds(zoff, _RPS)])

        @pl.when(s == _NS - 1)
        def _dtail():
            pltpu.sync_copy(acc_sh.at[pl.ds(_NS * _RPS, _RTAIL)],
                            agg2_hbm.at[c, pl.ds(_NS * _RPS, _RTAIL)])

    return agg


def _tc_pre_body(x_ref, w0_ref, w1_ref, degcnt_ref, g_ref, rs2_ref):
    deg = degcnt_ref[:_N2P] + degcnt_ref[_N2P:] + 1.0
    rs2 = lax.rsqrt(deg)
    rs2_ref[...] = rs2
    x = x_ref[...]
    h0 = jnp.dot(x, w0_ref[...], preferred_element_type=jnp.float32)
    h1 = jnp.dot(x, w1_ref[...], preferred_element_type=jnp.float32)
    g_ref[:_N] = h0 * rs2[:_N, None]
    g_ref[_N:] = h1 * rs2[_N:_N2, None]


def _bn_from(agg_ref, g_ref, rs2_ref, bsum_ref, gamma_ref, beta_ref):
    rs2 = rs2_ref[...]
    g = g_ref[...]
    pre = (agg_ref[0] + agg_ref[1]
           + g[:_N] * rs2[:_N, None]
           + g[_N:] * rs2[_N:_N2, None]
           + bsum_ref[...])
    mu = jnp.mean(pre, axis=0)
    var = jnp.mean((pre - mu) ** 2, axis=0)
    return gamma_ref[...] * (pre - mu) / jnp.sqrt(var + 1e-5) + beta_ref[...]


def _tc_post_body(agg_ref, g_ref, rs2_ref, bsum_ref, gamma_ref, beta_ref,
                  wn0_ref, wn1_ref, gout_ref):
    h = jnp.maximum(_bn_from(agg_ref, g_ref, rs2_ref, bsum_ref,
                             gamma_ref, beta_ref), 0.0)
    rs2 = rs2_ref[...]
    gout_ref[:_N] = jnp.dot(h, wn0_ref[...],
                            preferred_element_type=jnp.float32) * rs2[:_N, None]
    gout_ref[_N:] = jnp.dot(h, wn1_ref[...],
                            preferred_element_type=jnp.float32) * rs2[_N:_N2, None]


def _l2n(x):
    n = jnp.sqrt(jnp.sum(x * x, axis=1, keepdims=True))
    return x / jnp.maximum(n, 1e-12)


def _tc_heads_body(agg_ref, g_ref, rs2_ref, bsum_ref, gamma_ref, beta_ref,
                   we1_ref, be1_ref, we2_ref, be2_ref,
                   wp1a_ref, bp1a_ref, wp1b_ref, bp1b_ref,
                   wp2a_ref, bp2a_ref, wp2b_ref, bp2b_ref,
                   e1_ref, e2_ref, p1_ref, p2_ref):
    h = _bn_from(agg_ref, g_ref, rs2_ref, bsum_ref, gamma_ref, beta_ref)
    dot = lambda a, b: jnp.dot(a, b, preferred_element_type=jnp.float32)
    e1 = jnp.tanh(dot(h, we1_ref[...]) + be1_ref[...])
    e2 = _l2n(jnp.tanh(dot(h, we2_ref[...]) + be2_ref[...]))
    p1 = _l2n(dot(jnp.maximum(dot(e1, wp1a_ref[...]) + bp1a_ref[...], 0.0),
                  wp1b_ref[...]) + bp1b_ref[...])
    p2 = _l2n(dot(jnp.maximum(dot(e2, wp2a_ref[...]) + bp2a_ref[...], 0.0),
                  wp2b_ref[...]) + bp2b_ref[...])
    e1_ref[...] = e1
    e2_ref[...] = e2
    p1_ref[...] = p1
    p2_ref[...] = p2


def _tc_pre(x, w0, w1, degcnt, interpret=False):
    return pl.pallas_call(
        _tc_pre_body,
        out_shape=(_f32((_N2, _D)), _f32((_N2P,))),
        interpret=interpret,
    )(x, w0, w1, degcnt)


def _tc_post(agg2, g, rs2, bsum, gamma, beta, wn0, wn1, interpret=False):
    return pl.pallas_call(
        _tc_post_body,
        out_shape=_f32((_N2, _D)),
        interpret=interpret,
    )(agg2, g, rs2, bsum, gamma, beta, wn0, wn1)


def _tc_heads(agg2, g, rs2, bsum, gamma, beta, heads, interpret=False):
    return pl.pallas_call(
        _tc_heads_body,
        out_shape=(_f32((_N, _D)),) * 4,
        interpret=interpret,
    )(agg2, g, rs2, bsum, gamma, beta, *heads)


def kernel(x, edge_index, edge_type, Wconv, bconv, bn_gamma, bn_beta,
           W_e1, b_e1, W_e2, b_e2, Wp1a, bp1a, Wp1b, bp1b,
           Wp2a, bp2a, Wp2b, bp2b):
    src = edge_index[0]
    dst = edge_index[1]
    et = edge_type.astype(jnp.int32)
    ones = jnp.ones((_CH,), jnp.float32)
    zwords = jnp.zeros((_WPS,), jnp.float32)
    zrows = jnp.zeros((_RPS, _D), jnp.float32)

    sc_prep = _make_sc_prep()
    sc_sgather = _make_sc_sgather()
    sc_agg = _make_sc_agg()

    esrc2, edst2, degcnt = sc_prep(src, dst, et, ones, zwords)
    g, rs2 = _tc_pre(x, Wconv[0, 0], Wconv[0, 1], degcnt)
    sedge = sc_sgather(edst2, rs2)

    for layer in range(3):
        agg2 = sc_agg(g, esrc2, dst, sedge, zrows)
        bsum = bconv[layer, 0] + bconv[layer, 1]
        if layer < 2:
            g = _tc_post(agg2, g, rs2, bsum, bn_gamma[layer], bn_beta[layer],
                         Wconv[layer + 1, 0], Wconv[layer + 1, 1])
        else:
            heads = (W_e1, b_e1, W_e2, b_e2, Wp1a, bp1a, Wp1b, bp1b,
                     Wp2a, bp2a, Wp2b, bp2b)
            e1, e2, p1, p2 = _tc_heads(agg2, g, rs2, bsum,
                                       bn_gamma[layer], bn_beta[layer], heads)
    return (e1, e2, p1, p2)


# speedup vs baseline: 28.2235x; 1.2413x over previous
import functools

import jax
import jax.numpy as jnp
from jax import lax
from jax.experimental import pallas as pl
from jax.experimental.pallas import tpu as pltpu
from jax.experimental.pallas import tpu_sc as plsc

_N = 10000
_E = 320000
_D = 128
_NC = 2
_NS = 16
_NW = _NC * _NS
_EPT = _E // _NW
_CH = 128
_NFULL = _EPT // _CH
_REM = _EPT - _NFULL * _CH
_N2 = 2 * _N
_N2P = 20480
_WPS = _N2P // _NS
_RPS = 624
_RTAIL = _N - _RPS * _NS


def _f32(shape):
    return jax.ShapeDtypeStruct(shape, jnp.float32)


def _i32(shape):
    return jax.ShapeDtypeStruct(shape, jnp.int32)


def _make_sc_prep(interpret=False):
    mesh = plsc.VectorSubcoreMesh(core_axis_name="c", subcore_axis_name="s")

    @functools.partial(
        pl.kernel,
        out_type=(_i32((_E,)), _i32((_E,)), _f32((_NC * _N2P,))),
        mesh=mesh,
        scratch_types=(
            pltpu.VMEM((_CH,), jnp.int32),
            pltpu.VMEM((_CH,), jnp.int32),
            pltpu.VMEM((_CH,), jnp.int32),
            pltpu.VMEM((_CH,), jnp.int32),
            pltpu.VMEM((_CH,), jnp.int32),
            pltpu.VMEM((_CH,), jnp.int32),
            pltpu.VMEM((_CH,), jnp.int32),
            pltpu.VMEM((_CH,), jnp.int32),
            pltpu.VMEM((_CH,), jnp.int32),
            pltpu.VMEM((_CH,), jnp.int32),
            pltpu.VMEM((_CH,), jnp.int32),
            pltpu.VMEM((_CH,), jnp.int32),
            pltpu.VMEM((_CH,), jnp.int32),
            pltpu.VMEM((_CH,), jnp.int32),
            pltpu.VMEM((_CH,), jnp.int32),
            pltpu.VMEM((_CH,), jnp.float32),
            pltpu.SemaphoreType.DMA,
            pltpu.SemaphoreType.DMA,
            pltpu.SemaphoreType.DMA,
            pltpu.SemaphoreType.DMA,
            pltpu.SemaphoreType.DMA,
            pltpu.SemaphoreType.DMA,
            pltpu.VMEM((_REM,), jnp.int32),
            pltpu.VMEM((_REM,), jnp.int32),
            pltpu.VMEM((_REM,), jnp.int32),
            pltpu.VMEM((_REM,), jnp.int32),
            pltpu.VMEM((_REM,), jnp.int32),
            pltpu.VMEM((_REM,), jnp.float32),
            pltpu.VMEM_SHARED((_N2P,), jnp.float32),
        ),
        compiler_params=pltpu.CompilerParams(needs_layout_passes=False),
        interpret=interpret,
    )
    def prep(src_hbm, dst_hbm, et_hbm, ones_hbm, zw_hbm,
             esrc2_hbm, edst2_hbm, degcnt_hbm,
             src0, dst0, et0, es0, ed0,
             src1, dst1, et1, es1, ed1,
             src2, dst2, et2, es2, ed2,
             ones_v,
             isem0, isem1, isem2, osem0, osem1, osem2,
             src_r, dst_r, et_r, es_r, ed_r, ones_r,
             acc_sh):
        c = lax.axis_index("c")
        s = lax.axis_index("s")
        w = c * _NS + s
        zoff = pl.multiple_of(s * _WPS, 8)
        pltpu.sync_copy(zw_hbm, acc_sh.at[pl.ds(zoff, _WPS)])
        pltpu.sync_copy(ones_hbm, ones_v)
        pltpu.sync_copy(ones_hbm.at[pl.ds(0, _REM)], ones_r)
        plsc.subcore_barrier()

        base0 = w * _EPT
        bufs = ((src0, dst0, et0, es0, ed0, isem0, osem0),
                (src1, dst1, et1, es1, ed1, isem1, osem1),
                (src2, dst2, et2, es2, ed2, isem2, osem2))

        def coff(ci):
            return pl.multiple_of(base0 + ci * _CH, 8)

        def issue_in(off, b):
            sv, dv, tv, _, _, isem, _ = bufs[b]
            pltpu.async_copy(src_hbm.at[pl.ds(off, _CH)], sv, isem)
            pltpu.async_copy(dst_hbm.at[pl.ds(off, _CH)], dv, isem)
            pltpu.async_copy(et_hbm.at[pl.ds(off, _CH)], tv, isem)

        def wait_in(b):
            sv, dv, tv, _, _, isem, _ = bufs[b]
            z = pl.ds(0, _CH)
            pltpu.make_async_copy(src_hbm.at[z], sv, isem).wait()
            pltpu.make_async_copy(dst_hbm.at[z], dv, isem).wait()
            pltpu.make_async_copy(et_hbm.at[z], tv, isem).wait()

        def compute(b):
            sv, dv, tv, es, ed, _, _ = bufs[b]
            for j in range(_CH // 16):
                sl = pl.ds(16 * j, 16)
                t16 = tv[sl]
                es[sl] = t16 * _N + sv[sl]
                ed[sl] = t16 * _N + dv[sl]

        def do_out(off, b):
            _, _, _, es, ed, _, _ = bufs[b]
            pltpu.sync_copy(es, esrc2_hbm.at[pl.ds(off, _CH)])
            pltpu.sync_copy(ed, edst2_hbm.at[pl.ds(off, _CH)])
            pltpu.sync_copy(ones_v, acc_sh.at[ed], add=True)

        n3 = _NFULL // 3
        issue_in(coff(0), 0)
        issue_in(coff(1), 1)

        @pl.loop(0, n3)
        def _outer(g3):
            for b in range(3):
                ci = g3 * 3 + b
                b2 = (b + 2) % 3
                if b == 0:
                    issue_in(coff(ci + 2), b2)
                else:
                    @pl.when(g3 < n3 - 1)
                    def _w1():
                        issue_in(coff(ci + 2), b2)
                wait_in(b)
                compute(b)
                do_out(coff(ci), b)

        roff = pl.multiple_of(base0 + _NFULL * _CH, 8)
        pltpu.sync_copy(src_hbm.at[pl.ds(roff, _REM)], src_r)
        pltpu.sync_copy(dst_hbm.at[pl.ds(roff, _REM)], dst_r)
        pltpu.sync_copy(et_hbm.at[pl.ds(roff, _REM)], et_r)
        t16 = et_r[...]
        es_r[...] = t16 * _N + src_r[...]
        ed_r[...] = t16 * _N + dst_r[...]
        pltpu.sync_copy(es_r, esrc2_hbm.at[pl.ds(roff, _REM)])
        pltpu.sync_copy(ed_r, edst2_hbm.at[pl.ds(roff, _REM)])
        pltpu.sync_copy(ones_r, acc_sh.at[ed_r], add=True)

        plsc.subcore_barrier()
        doff = pl.multiple_of(s * _WPS, 8)
        doff2 = pl.multiple_of(c * _N2P + s * _WPS, 8)
        pltpu.sync_copy(acc_sh.at[pl.ds(doff, _WPS)],
                        degcnt_hbm.at[pl.ds(doff2, _WPS)])

    return prep


def _make_sc_sgather(interpret=False):
    mesh = plsc.VectorSubcoreMesh(core_axis_name="c", subcore_axis_name="s")

    @functools.partial(
        pl.kernel,
        out_type=_f32((_E,)),
        mesh=mesh,
        scratch_types=(
            pltpu.VMEM((_N2P,), jnp.float32),
            pltpu.VMEM((_CH,), jnp.int32),
            pltpu.VMEM((_CH,), jnp.float32),
            pltpu.VMEM((_CH,), jnp.int32),
            pltpu.VMEM((_CH,), jnp.float32),
            pltpu.VMEM((_CH,), jnp.int32),
            pltpu.VMEM((_CH,), jnp.float32),
            pltpu.SemaphoreType.DMA,
            pltpu.SemaphoreType.DMA,
            pltpu.SemaphoreType.DMA,
            pltpu.SemaphoreType.DMA,
            pltpu.SemaphoreType.DMA,
            pltpu.SemaphoreType.DMA,
        ),
        compiler_params=pltpu.CompilerParams(needs_layout_passes=False),
        interpret=interpret,
    )
    def sgather(edst2_hbm, rs2_hbm, s_hbm, rs_v,
                ei0, sv0, ei1, sv1, ei2, sv2,
                isem0, isem1, isem2, osem0, osem1, osem2):
        c = lax.axis_index("c")
        s = lax.axis_index("s")
        w = c * _NS + s
        pltpu.sync_copy(rs2_hbm, rs_v)
        base0 = w * _EPT
        bufs = ((ei0, sv0, isem0, osem0),
                (ei1, sv1, isem1, osem1),
                (ei2, sv2, isem2, osem2))

        def coff(ci):
            return pl.multiple_of(base0 + ci * _CH, 8)

        def issue_in(off, b):
            ei, _, isem, _ = bufs[b]
            pltpu.async_copy(edst2_hbm.at[pl.ds(off, _CH)], ei, isem)

        def wait_in(b):
            ei, _, isem, _ = bufs[b]
            pltpu.make_async_copy(edst2_hbm.at[pl.ds(0, _CH)], ei, isem).wait()

        def compute(b):
            ei, sv, _, _ = bufs[b]
            for jg in range(_CH // 16):
                sl = pl.ds(16 * jg, 16)
                sv[sl] = plsc.load_gather(rs_v, [ei[sl]])

        n3 = _NFULL // 3
        issue_in(coff(0), 0)
        issue_in(coff(1), 1)

        @pl.loop(0, n3)
        def _outer(g3):
            for b in range(3):
                ci = g3 * 3 + b
                b2 = (b + 2) % 3
                if b == 0:
                    issue_in(coff(ci + 2), b2)
                else:
                    @pl.when(g3 < n3 - 1)
                    def _w1():
                        issue_in(coff(ci + 2), b2)
                wait_in(b)
                compute(b)
                pltpu.sync_copy(bufs[b][1], s_hbm.at[pl.ds(coff(ci), _CH)])

        roff = pl.multiple_of(base0 + _EPT - _CH, 8)
        pltpu.sync_copy(edst2_hbm.at[pl.ds(roff, _CH)], ei0)
        compute(0)
        pltpu.sync_copy(sv0, s_hbm.at[pl.ds(roff, _CH)])

    return sgather


def _make_sc_agg(interpret=False):
    mesh = plsc.VectorSubcoreMesh(core_axis_name="c", subcore_axis_name="s")

    @functools.partial(
        pl.kernel,
        out_type=_f32((_NC, _N, _D)),
        mesh=mesh,
        scratch_types=(
            pltpu.VMEM((_CH,), jnp.int32),
            pltpu.VMEM((_CH,), jnp.int32),
            pltpu.VMEM((_CH,), jnp.float32),
            pltpu.VMEM((_CH, _D), jnp.float32),
            pltpu.VMEM((_CH,), jnp.int32),
            pltpu.VMEM((_CH,), jnp.int32),
            pltpu.VMEM((_CH,), jnp.float32),
            pltpu.VMEM((_CH, _D), jnp.float32),
            pltpu.VMEM((_CH,), jnp.int32),
            pltpu.VMEM((_CH,), jnp.int32),
            pltpu.VMEM((_CH,), jnp.float32),
            pltpu.VMEM((_CH, _D), jnp.float32),
            pltpu.SemaphoreType.DMA,
            pltpu.SemaphoreType.DMA,
            pltpu.SemaphoreType.DMA,
            pltpu.SemaphoreType.DMA,
            pltpu.SemaphoreType.DMA,
            pltpu.SemaphoreType.DMA,
            pltpu.SemaphoreType.DMA,
            pltpu.SemaphoreType.DMA,
            pltpu.SemaphoreType.DMA,
            pltpu.VMEM_SHARED((_N, _D), jnp.float32),
        ),
        compiler_params=pltpu.CompilerParams(needs_layout_passes=False),
        interpret=interpret,
    )
    def agg(g_hbm, esrc2_hbm, dst_hbm, s_hbm, zr_hbm,
            agg2_hbm,
            gi0, di0, sv0, rows0,
            gi1, di1, sv1, rows1,
            gi2, di2, sv2, rows2,
            gsem0, gsem1, gsem2, ssem0, ssem1, ssem2,
            isem0, isem1, isem2,
            acc_sh):
        c = lax.axis_index("c")
        s = lax.axis_index("s")
        w = c * _NS + s
        zoff = pl.multiple_of(s * _RPS, 8)
        pltpu.sync_copy(zr_hbm, acc_sh.at[pl.ds(zoff, _RPS)])

        @pl.when(s == _NS - 1)
        def _ztail():
            pltpu.sync_copy(zr_hbm.at[pl.ds(0, _RTAIL)],
                            acc_sh.at[pl.ds(_NS * _RPS, _RTAIL)])

        plsc.subcore_barrier()

        base0 = w * _EPT
        bufs = ((gi0, di0, sv0, rows0, gsem0, ssem0, isem0),
                (gi1, di1, sv1, rows1, gsem1, ssem1, isem1),
                (gi2, di2, sv2, rows2, gsem2, ssem2, isem2))

        def issue_idx(off, b):
            gi, di, sv, _, _, _, isem = bufs[b]
            pltpu.async_copy(esrc2_hbm.at[pl.ds(off, _CH)], gi, isem)
            pltpu.async_copy(dst_hbm.at[pl.ds(off, _CH)], di, isem)
            pltpu.async_copy(s_hbm.at[pl.ds(off, _CH)], sv, isem)

        def wait_idx(b):
            gi, di, sv, _, _, _, isem = bufs[b]
            z = pl.ds(0, _CH)
            pltpu.make_async_copy(esrc2_hbm.at[z], gi, isem).wait()
            pltpu.make_async_copy(dst_hbm.at[z], di, isem).wait()
            pltpu.make_async_copy(s_hbm.at[z], sv, isem).wait()

        def start_gather(b):
            gi, _, _, rows, gsem, _, _ = bufs[b]
            pltpu.async_copy(g_hbm.at[gi], rows, gsem)

        def wait_gather(b):
            gi, _, _, rows, gsem, _, _ = bufs[b]
            pltpu.make_async_copy(g_hbm.at[gi], rows, gsem).wait()

        def wait_scatter(b):
            _, di, _, rows, _, ssem, _ = bufs[b]
            pltpu.make_async_copy(rows, acc_sh.at[di], ssem).wait()

        def scale(b):
            _, _, sv, rows, _, _, _ = bufs[b]

            @pl.loop(0, _CH // 16)
            def _grp(jg):
                off16 = pl.multiple_of(jg * 16, 16)
                svals = sv[pl.ds(off16, 16)]
                for l in range(16):
                    splat = svals.at[jnp.full((16,), l, jnp.int32)].get(
                        mode='promise_in_bounds')
                    for j in range(_D // 16):
                        sl = pl.ds(16 * j, 16)
                        rows[off16 + l, sl] = rows[off16 + l, sl] * splat

        def start_scatter(b):
            _, di, _, rows, _, ssem, _ = bufs[b]
            pltpu.async_copy(rows, acc_sh.at[di], ssem, add=True)

        def coff(ci):
            return pl.multiple_of(base0 + ci * _CH, 8)

        n3 = _NFULL // 3
        issue_idx(coff(0), 0)
        issue_idx(coff(1), 1)
        wait_idx(0)
        start_gather(0)

        @pl.loop(0, n3)
        def _outer(g3):
            for b in range(3):
                ci = g3 * 3 + b
                b1 = (b + 1) % 3
                b2 = (b + 2) % 3
                if b == 0:
                    @pl.when(g3 >= 1)
                    def _w0():
                        wait_scatter(b2)
                    issue_idx(coff(ci + 2), b2)
                else:
                    @pl.when(g3 < n3 - 1)
                    def _w1():
                        wait_scatter(b2)
                        issue_idx(coff(ci + 2), b2)
                if b < 2:
                    wait_idx(b1)
                    start_gather(b1)
                else:
                    @pl.when(g3 < n3 - 1)
                    def _g1():
                        wait_idx(b1)
                        start_gather(b1)
                wait_gather(b)
                scale(b)
                start_scatter(b)

        wait_scatter(0)
        wait_scatter(1)
        wait_scatter(2)

        roff = pl.multiple_of(base0 + _EPT - _CH, 8)
        issue_idx(roff, 0)
        wait_idx(0)
        for jg in range((_CH - _REM) // 16):
            sv0[pl.ds(16 * jg, 16)] = jnp.zeros((16,), jnp.float32)
        pltpu.sync_copy(g_hbm.at[gi0], rows0)
        scale(0)
        pltpu.sync_copy(rows0, acc_sh.at[di0], add=True)

        plsc.subcore_barrier()
        pltpu.sync_copy(acc_sh.at[pl.ds(zoff, _RPS)],
                        agg2_hbm.at[c, pl.ds(zoff, _RPS)])

        @pl.when(s == _NS - 1)
        def _dtail():
            pltpu.sync_copy(acc_sh.at[pl.ds(_NS * _RPS, _RTAIL)],
                            agg2_hbm.at[c, pl.ds(_NS * _RPS, _RTAIL)])

    return agg


def _tc_pre_body(x_ref, w0_ref, w1_ref, degcnt_ref, g_ref, rs2_ref):
    deg = degcnt_ref[:_N2P] + degcnt_ref[_N2P:] + 1.0
    rs2 = lax.rsqrt(deg)
    rs2_ref[...] = rs2
    x = x_ref[...]
    h0 = jnp.dot(x, w0_ref[...], preferred_element_type=jnp.float32)
    h1 = jnp.dot(x, w1_ref[...], preferred_element_type=jnp.float32)
    g_ref[:_N] = h0 * rs2[:_N, None]
    g_ref[_N:] = h1 * rs2[_N:_N2, None]


def _bn_from(agg_ref, g_ref, rs2_ref, bsum_ref, gamma_ref, beta_ref):
    rs2 = rs2_ref[...]
    g = g_ref[...]
    pre = (agg_ref[0] + agg_ref[1]
           + g[:_N] * rs2[:_N, None]
           + g[_N:] * rs2[_N:_N2, None]
           + bsum_ref[...])
    mu = jnp.mean(pre, axis=0)
    var = jnp.mean((pre - mu) ** 2, axis=0)
    return gamma_ref[...] * (pre - mu) / jnp.sqrt(var + 1e-5) + beta_ref[...]


def _tc_post_body(agg_ref, g_ref, rs2_ref, bsum_ref, gamma_ref, beta_ref,
                  wn0_ref, wn1_ref, gout_ref):
    h = jnp.maximum(_bn_from(agg_ref, g_ref, rs2_ref, bsum_ref,
                             gamma_ref, beta_ref), 0.0)
    rs2 = rs2_ref[...]
    gout_ref[:_N] = jnp.dot(h, wn0_ref[...],
                            preferred_element_type=jnp.float32) * rs2[:_N, None]
    gout_ref[_N:] = jnp.dot(h, wn1_ref[...],
                            preferred_element_type=jnp.float32) * rs2[_N:_N2, None]


def _l2n(x):
    n = jnp.sqrt(jnp.sum(x * x, axis=1, keepdims=True))
    return x / jnp.maximum(n, 1e-12)


def _tc_heads_body(agg_ref, g_ref, rs2_ref, bsum_ref, gamma_ref, beta_ref,
                   we1_ref, be1_ref, we2_ref, be2_ref,
                   wp1a_ref, bp1a_ref, wp1b_ref, bp1b_ref,
                   wp2a_ref, bp2a_ref, wp2b_ref, bp2b_ref,
                   e1_ref, e2_ref, p1_ref, p2_ref):
    h = _bn_from(agg_ref, g_ref, rs2_ref, bsum_ref, gamma_ref, beta_ref)
    dot = lambda a, b: jnp.dot(a, b, preferred_element_type=jnp.float32)
    e1 = jnp.tanh(dot(h, we1_ref[...]) + be1_ref[...])
    e2 = _l2n(jnp.tanh(dot(h, we2_ref[...]) + be2_ref[...]))
    p1 = _l2n(dot(jnp.maximum(dot(e1, wp1a_ref[...]) + bp1a_ref[...], 0.0),
                  wp1b_ref[...]) + bp1b_ref[...])
    p2 = _l2n(dot(jnp.maximum(dot(e2, wp2a_ref[...]) + bp2a_ref[...], 0.0),
                  wp2b_ref[...]) + bp2b_ref[...])
    e1_ref[...] = e1
    e2_ref[...] = e2
    p1_ref[...] = p1
    p2_ref[...] = p2


def _tc_pre(x, w0, w1, degcnt, interpret=False):
    return pl.pallas_call(
        _tc_pre_body,
        out_shape=(_f32((_N2, _D)), _f32((_N2P,))),
        interpret=interpret,
    )(x, w0, w1, degcnt)


def _tc_post(agg2, g, rs2, bsum, gamma, beta, wn0, wn1, interpret=False):
    return pl.pallas_call(
        _tc_post_body,
        out_shape=_f32((_N2, _D)),
        interpret=interpret,
    )(agg2, g, rs2, bsum, gamma, beta, wn0, wn1)


def _tc_heads(agg2, g, rs2, bsum, gamma, beta, heads, interpret=False):
    return pl.pallas_call(
        _tc_heads_body,
        out_shape=(_f32((_N, _D)),) * 4,
        interpret=interpret,
    )(agg2, g, rs2, bsum, gamma, beta, *heads)


def kernel(x, edge_index, edge_type, Wconv, bconv, bn_gamma, bn_beta,
           W_e1, b_e1, W_e2, b_e2, Wp1a, bp1a, Wp1b, bp1b,
           Wp2a, bp2a, Wp2b, bp2b):
    src = edge_index[0]
    dst = edge_index[1]
    et = edge_type.astype(jnp.int32)
    ones = jnp.ones((_CH,), jnp.float32)
    zwords = jnp.zeros((_WPS,), jnp.float32)
    zrows = jnp.zeros((_RPS, _D), jnp.float32)

    sc_prep = _make_sc_prep()
    sc_sgather = _make_sc_sgather()
    sc_agg = _make_sc_agg()

    esrc2, edst2, degcnt = sc_prep(src, dst, et, ones, zwords)
    g, rs2 = _tc_pre(x, Wconv[0, 0], Wconv[0, 1], degcnt)
    sedge = sc_sgather(edst2, rs2)

    for layer in range(3):
        agg2 = sc_agg(g, esrc2, dst, sedge, zrows)
        bsum = bconv[layer, 0] + bconv[layer, 1]
        if layer < 2:
            g = _tc_post(agg2, g, rs2, bsum, bn_gamma[layer], bn_beta[layer],
                         Wconv[layer + 1, 0], Wconv[layer + 1, 1])
        else:
            heads = (W_e1, b_e1, W_e2, b_e2, Wp1a, bp1a, Wp1b, bp1b,
                     Wp2a, bp2a, Wp2b, bp2b)
            e1, e2, p1, p2 = _tc_heads(agg2, g, rs2, bsum,
                                       bn_gamma[layer], bn_beta[layer], heads)
    return (e1, e2, p1, p2)
